# Initial kernel scaffold; baseline (speedup 1.0000x reference)
import jax
import jax.numpy as jnp
from jax.experimental import pallas as pl


def kernel(x, atom_types, seq_neighs, weight):
    raise NotImplementedError("write your pallas kernel here")


# speedup vs baseline: 78.6996x; 78.6996x over previous
import functools

import jax
import jax.numpy as jnp
from jax import lax
from jax.experimental import pallas as pl
from jax.experimental.pallas import tpu as pltpu
from jax.experimental.pallas import tpu_sc as plsc

_NUM_CORES = 2
_NUM_SUBCORES = 16
_LANES = 16
_NW = _NUM_CORES * _NUM_SUBCORES

_GATHER_CHUNK = 128


def _sc_body(n_nodes, n_edges, b_pad, f_dim,
             x_hbm, atom_hbm, seq_hbm, partials_hbm, g_hbm,
             idx_v, rows_v, src_v, dst_v, hist_v, gsem, hsem):
    ew = n_edges // _NW
    bw = b_pad // _NW
    wid = lax.axis_index("s") * _NUM_CORES + lax.axis_index("c")

    gbase = wid * bw
    pltpu.sync_copy(atom_hbm.at[pl.ds(gbase, bw)], idx_v)
    n_chunks = bw // _GATHER_CHUNK
    for ci in range(n_chunks):
        sl = pl.ds(ci * _GATHER_CHUNK, _GATHER_CHUNK)
        pltpu.async_copy(x_hbm.at[idx_v.at[sl]], rows_v.at[sl], gsem)

    ebase = wid * ew
    pltpu.sync_copy(seq_hbm.at[pl.ds(ebase, ew)], src_v)
    pltpu.sync_copy(seq_hbm.at[pl.ds(n_edges + ebase, ew)], dst_v)

    zeros = jnp.zeros((_LANES,), jnp.float32)

    @pl.loop(0, 3 * n_nodes, step=_LANES)
    def _(i):
        hist_v[pl.ds(i, _LANES)] = zeros

    @pl.loop(0, ew, step=_LANES)
    def _(i):
        s = src_v[pl.ds(i, _LANES)]
        d = dst_v[pl.ds(i, _LANES)]
        bins = (d - s + 1) * n_nodes + s
        cnt, last = plsc.scan_count(bins)
        plsc.addupdate_scatter(hist_v, [bins], cnt.astype(jnp.float32),
                               mask=last)

    for k in range(3):
        pltpu.async_copy(hist_v.at[pl.ds(k * n_nodes, n_nodes)],
                         partials_hbm.at[pl.ds((k * _NW + wid) * n_nodes,
                                               n_nodes)], hsem)

    for ci in range(n_chunks):
        sl = pl.ds(ci * _GATHER_CHUNK, _GATHER_CHUNK)
        pltpu.make_async_copy(x_hbm.at[idx_v.at[sl]], rows_v.at[sl],
                              gsem).wait()
    pltpu.sync_copy(rows_v, g_hbm.at[pl.ds(gbase, bw)])

    for k in range(3):
        pltpu.make_async_copy(hist_v.at[pl.ds(k * n_nodes, n_nodes)],
                              partials_hbm.at[pl.ds((k * _NW + wid) * n_nodes,
                                                    n_nodes)], hsem).wait()


def _tc_body(n_nodes, partials_ref, g_ref, w_ref, out_ref):
    counts = jnp.sum(partials_ref[...], axis=1)
    coef = lax.dot_general(
        counts, w_ref[...],
        dimension_numbers=(((0,), (0,)), ((), ())),
        preferred_element_type=jnp.float32,
    )
    coef = coef + w_ref[1, :][None, :]
    out_ref[...] = coef * g_ref[0:n_nodes, :]


def kernel(x, atom_types, seq_neighs, weight):
    n_nodes, f_dim = x.shape
    n_edges = seq_neighs.shape[1]
    align = 8 * _NW * _GATHER_CHUNK // _GATHER_CHUNK
    b_pad = ((n_nodes + _NW * _GATHER_CHUNK - 1)
             // (_NW * _GATHER_CHUNK)) * (_NW * _GATHER_CHUNK)
    atom_pad = jnp.concatenate(
        [atom_types.astype(jnp.int32),
         jnp.zeros((b_pad - n_nodes,), jnp.int32)])

    mesh = plsc.VectorSubcoreMesh(core_axis_name="c", subcore_axis_name="s")
    bw = b_pad // _NW
    sc = pl.kernel(
        functools.partial(_sc_body, n_nodes, n_edges, b_pad, f_dim),
        out_type=(
            jax.ShapeDtypeStruct((3 * _NW * n_nodes,), jnp.float32),
            jax.ShapeDtypeStruct((b_pad, f_dim), jnp.float32),
        ),
        mesh=mesh,
        scratch_types=[
            pltpu.VMEM((bw,), jnp.int32),
            pltpu.VMEM((bw, f_dim), jnp.float32),
            pltpu.VMEM((n_edges // _NW,), jnp.int32),
            pltpu.VMEM((n_edges // _NW,), jnp.int32),
            pltpu.VMEM((3 * n_nodes,), jnp.float32),
            pltpu.SemaphoreType.DMA,
            pltpu.SemaphoreType.DMA,
        ],
        compiler_params=pltpu.CompilerParams(needs_layout_passes=False),
    )
    partials, g = sc(x, atom_pad, seq_neighs.astype(jnp.int32).reshape(-1))
    partials = partials.reshape(3, _NW, n_nodes)

    out = pl.pallas_call(
        functools.partial(_tc_body, n_nodes),
        out_shape=jax.ShapeDtypeStruct((n_nodes, f_dim), jnp.float32),
    )(partials, g, weight)
    return out
